# baseline (speedup 1.0000x reference)
import functools

import jax
import jax.numpy as jnp
from jax import lax
from jax.experimental import pallas as pl
from jax.experimental.pallas import tpu as pltpu
from jax.experimental.pallas import tpu_sc as plsc

N = 10000
E = 320000
C = 128
NC = 2
NS = 16
NW = NC * NS
K = 128
NCHUNK = 80
EPT = NCHUNK * K
EPAD = NW * EPT
ACC_N = 10112
RPT = ACC_N // NS
DEG_EPT = E // NW

_MESH = plsc.VectorSubcoreMesh(
    core_axis_name="c", subcore_axis_name="s", num_cores=NC, num_subcores=NS
)


@functools.partial(
    pl.kernel,
    out_type=jax.ShapeDtypeStruct((NW * N,), jnp.float32),
    mesh=_MESH,
    scratch_types=[
        pltpu.VMEM((DEG_EPT,), jnp.int32),
        pltpu.VMEM((N,), jnp.float32),
    ],
    compiler_params=pltpu.CompilerParams(needs_layout_passes=False),
)
def _deg_kernel(dst_hbm, zeros_hbm, out_hbm, dstbuf, deg):
    wid = lax.axis_index("c") * NS + lax.axis_index("s")
    pltpu.sync_copy(dst_hbm.at[pl.ds(wid * DEG_EPT, DEG_EPT)], dstbuf)
    pltpu.sync_copy(zeros_hbm, deg)
    ones = jnp.ones((16,), jnp.float32)

    def body(i, _):
        idx = dstbuf[pl.ds(i * 16, 16)]
        plsc.addupdate_scatter(deg, [idx], ones)
        return 0

    lax.fori_loop(0, DEG_EPT // 16, body, 0)
    pltpu.sync_copy(deg, out_hbm.at[pl.ds(wid * N, N)])


def _scale_body(parts_ref, x_ref, x2_ref, dinv_ref):
    parts = parts_ref[...]
    ones = jnp.ones((NW, 1), jnp.float32)
    deg = lax.dot_general(parts, ones, (((1,), (0,)), ((), ())),
                          preferred_element_type=jnp.float32)
    dinv = lax.rsqrt(deg + 1.0)
    x2_ref[...] = x_ref[...] * dinv
    dinv_ref[...] = dinv


def _scale_call(parts_t, x):
    BR = 2000
    return pl.pallas_call(
        _scale_body,
        grid=(N // BR,),
        in_specs=[
            pl.BlockSpec((BR, NW), lambda i: (i, 0)),
            pl.BlockSpec((BR, C), lambda i: (i, 0)),
        ],
        out_specs=[
            pl.BlockSpec((BR, C), lambda i: (i, 0)),
            pl.BlockSpec((BR, 1), lambda i: (i, 0)),
        ],
        out_shape=[
            jax.ShapeDtypeStruct((N, C), jnp.float32),
            jax.ShapeDtypeStruct((N, 1), jnp.float32),
        ],
    )(parts_t, x)


@functools.partial(
    pl.kernel,
    out_type=jax.ShapeDtypeStruct((NC * ACC_N, C), jnp.float32),
    mesh=_MESH,
    scratch_types=[
        pltpu.VMEM((NCHUNK // 2, K), jnp.int32),
        pltpu.VMEM((NCHUNK // 2, K), jnp.int32),
        pltpu.VMEM((K, C), jnp.float32),
        pltpu.VMEM((K, C), jnp.float32),
        pltpu.VMEM_SHARED((ACC_N, C), jnp.float32),
        pltpu.SemaphoreType.DMA,
        pltpu.SemaphoreType.DMA,
        pltpu.SemaphoreType.DMA,
        pltpu.SemaphoreType.DMA,
    ],
    compiler_params=pltpu.CompilerParams(needs_layout_passes=False),
)
def _agg_kernel(src_hbm, dst_hbm, x2_hbm, zeros_hbm, out_hbm,
                srcbuf, dstbuf, rows0, rows1, acc, sem0, sem1, ssem0, ssem1):
    c = lax.axis_index("c")
    s = lax.axis_index("s")
    wid = c * NS + s
    BCH = NCHUNK // 2
    pltpu.sync_copy(zeros_hbm, acc.at[pl.ds(s * RPT, RPT)])
    plsc.subcore_barrier()

    def g_start(j, buf, sem):
        pltpu.async_copy(x2_hbm.at[srcbuf.at[j]], buf, sem)

    def g_wait(j, buf, sem):
        pltpu.make_async_copy(x2_hbm.at[srcbuf.at[j]], buf, sem).wait()

    def s_start(j, buf, sem):
        pltpu.async_copy(buf, acc.at[dstbuf.at[j]], sem, add=True)

    def s_wait(j, buf, sem):
        pltpu.make_async_copy(buf, acc.at[dstbuf.at[j]], sem).wait()

    def body(g, _):
        j0 = 2 * g
        j1 = j0 + 1
        g_wait(j0, rows0, sem0)
        s_start(j0, rows0, ssem0)
        g_wait(j1, rows1, sem1)
        s_start(j1, rows1, ssem1)
        s_wait(j0, rows0, ssem0)
        s_wait(j1, rows1, ssem1)

        @pl.when(g < BCH // 2 - 1)
        def _():
            g_start(j0 + 2, rows0, sem0)
            g_start(j1 + 2, rows1, sem1)

        return 0

    for h in range(NCHUNK // BCH):
        pltpu.sync_copy(src_hbm.at[wid, pl.ds(h * BCH, BCH)], srcbuf)
        pltpu.sync_copy(dst_hbm.at[wid, pl.ds(h * BCH, BCH)], dstbuf)
        g_start(0, rows0, sem0)
        g_start(1, rows1, sem1)
        lax.fori_loop(0, BCH // 2, body, 0)

    plsc.subcore_barrier()
    pltpu.sync_copy(acc.at[pl.ds(s * RPT, RPT)],
                    out_hbm.at[pl.ds(c * ACC_N + s * RPT, RPT)])


def _combine_body(parts_ref, x2_ref, dinv_ref, wt_ref, b_ref, out_ref):
    acc = parts_ref[0] + parts_ref[1] + x2_ref[...]
    pre = acc * dinv_ref[...]
    h = jnp.dot(pre, wt_ref[...], preferred_element_type=jnp.float32)
    o = h + b_ref[...]
    out_ref[...] = jnp.where(o >= 0, o, 0.01 * o) + 1.0


def _combine_call(parts2, x2, dinv, wt, b2):
    BR = 2000
    return pl.pallas_call(
        _combine_body,
        grid=(N // BR,),
        in_specs=[
            pl.BlockSpec((NC, BR, C), lambda i: (0, i, 0)),
            pl.BlockSpec((BR, C), lambda i: (i, 0)),
            pl.BlockSpec((BR, 1), lambda i: (i, 0)),
            pl.BlockSpec((C, C), lambda i: (0, 0)),
            pl.BlockSpec((1, C), lambda i: (0, 0)),
        ],
        out_specs=pl.BlockSpec((BR, C), lambda i: (i, 0)),
        out_shape=jax.ShapeDtypeStruct((N, C), jnp.float32),
    )(parts2, x2, dinv, wt, b2)


def kernel(x, edge_index, W, b):
    src = edge_index[0].astype(jnp.int32)
    dst = edge_index[1].astype(jnp.int32)
    zeros2d = jnp.zeros((RPT, C), jnp.float32)
    zeros1d = jnp.zeros((N,), jnp.float32)

    deg_parts = _deg_kernel(dst, zeros1d).reshape(NW, N)
    x2, dinv = _scale_call(deg_parts.T, x)
    npad = EPAD - E
    src_p = jnp.concatenate([src, jnp.zeros((npad,), jnp.int32)])
    dst_p = jnp.concatenate([dst, jnp.full((npad,), N, jnp.int32)])
    parts = _agg_kernel(src_p.reshape(NW, NCHUNK, K),
                        dst_p.reshape(NW, NCHUNK, K), x2, zeros2d)
    parts2 = parts.reshape(NC, ACC_N, C)
    out = _combine_call(parts2, x2, dinv, W.T, b.reshape(1, C))
    return out


# speedup vs baseline: 2.6295x; 2.6295x over previous
import functools

import jax
import jax.numpy as jnp
from jax import lax
from jax.experimental import pallas as pl
from jax.experimental.pallas import tpu as pltpu
from jax.experimental.pallas import tpu_sc as plsc

N = 10000
E = 320000
C = 128
NC = 2
NS = 16
NW = NC * NS
EPT = E // NW
K = 80
NCHUNK = EPT // K
ACC_N = 10240
RPT = ACC_N // NS

_MESH = plsc.VectorSubcoreMesh(
    core_axis_name="c", subcore_axis_name="s", num_cores=NC, num_subcores=NS
)


@functools.partial(
    pl.kernel,
    out_type=jax.ShapeDtypeStruct((NW * N,), jnp.float32),
    mesh=_MESH,
    scratch_types=[
        pltpu.VMEM((EPT,), jnp.int32),
        pltpu.VMEM((N,), jnp.float32),
    ],
    compiler_params=pltpu.CompilerParams(needs_layout_passes=False),
)
def _deg_kernel(dst_hbm, zeros_hbm, out_hbm, dstbuf, deg):
    wid = lax.axis_index("c") * NS + lax.axis_index("s")
    pltpu.sync_copy(dst_hbm.at[pl.ds(wid * EPT, EPT)], dstbuf)
    pltpu.sync_copy(zeros_hbm, deg)
    ones = jnp.ones((16,), jnp.float32)

    def body(i, _):
        idx = dstbuf[pl.ds(i * 16, 16)]
        plsc.addupdate_scatter(deg, [idx], ones)
        return 0

    lax.fori_loop(0, EPT // 16, body, 0)
    pltpu.sync_copy(deg, out_hbm.at[pl.ds(wid * N, N)])


def _scale_body(parts_ref, x_ref, x2_ref, dinv_ref):
    parts = parts_ref[...]
    ones = jnp.ones((NW, 1), jnp.float32)
    deg = lax.dot_general(parts, ones, (((1,), (0,)), ((), ())),
                          preferred_element_type=jnp.float32)
    dinv = lax.rsqrt(deg + 1.0)
    x2_ref[...] = x_ref[...] * dinv
    dinv_ref[...] = dinv


def _scale_call(parts_t, x):
    BR = 2000
    return pl.pallas_call(
        _scale_body,
        grid=(N // BR,),
        in_specs=[
            pl.BlockSpec((BR, NW), lambda i: (i, 0)),
            pl.BlockSpec((BR, C), lambda i: (i, 0)),
        ],
        out_specs=[
            pl.BlockSpec((BR, C), lambda i: (i, 0)),
            pl.BlockSpec((BR, 1), lambda i: (i, 0)),
        ],
        out_shape=[
            jax.ShapeDtypeStruct((N, C), jnp.float32),
            jax.ShapeDtypeStruct((N, 1), jnp.float32),
        ],
    )(parts_t, x)


@functools.partial(
    pl.kernel,
    out_type=jax.ShapeDtypeStruct((NC * ACC_N, C), jnp.float32),
    mesh=_MESH,
    scratch_types=[
        pltpu.VMEM((EPT,), jnp.int32),
        pltpu.VMEM((NCHUNK, K), jnp.int32),
        pltpu.VMEM((K, C), jnp.float32),
        pltpu.VMEM((K, C), jnp.float32),
        pltpu.VMEM_SHARED((ACC_N, C), jnp.float32),
        pltpu.SemaphoreType.DMA,
        pltpu.SemaphoreType.DMA,
        pltpu.SemaphoreType.DMA,
        pltpu.SemaphoreType.DMA,
    ],
    compiler_params=pltpu.CompilerParams(needs_layout_passes=False),
)
def _agg_kernel(src_hbm, dst_hbm, x2_hbm, zeros_hbm, out_hbm,
                srcbuf, dstbuf, rows0, rows1, acc, sem0, sem1, ssem0, ssem1):
    c = lax.axis_index("c")
    s = lax.axis_index("s")
    wid = c * NS + s
    pltpu.sync_copy(src_hbm.at[pl.ds(wid * EPT, EPT)], srcbuf)
    pltpu.sync_copy(dst_hbm.at[wid], dstbuf)
    pltpu.sync_copy(zeros_hbm, acc.at[pl.ds(s * RPT, RPT)])
    plsc.subcore_barrier()

    def g_start(i, buf, sem):
        pltpu.async_copy(x2_hbm.at[srcbuf.at[pl.ds(i * K, K)]], buf, sem)

    def g_wait(i, buf, sem):
        pltpu.make_async_copy(
            x2_hbm.at[srcbuf.at[pl.ds(i * K, K)]], buf, sem).wait()

    def s_start(i, buf, sem):
        pltpu.async_copy(buf, acc.at[dstbuf.at[i]], sem, add=True)

    def s_wait(i, buf, sem):
        pltpu.make_async_copy(buf, acc.at[dstbuf.at[i]], sem).wait()

    g_start(0, rows0, sem0)
    g_start(1, rows1, sem1)

    def body(g, _):
        i0 = 2 * g
        i1 = i0 + 1
        g_wait(i0, rows0, sem0)
        s_start(i0, rows0, ssem0)
        g_wait(i1, rows1, sem1)
        s_start(i1, rows1, ssem1)
        s_wait(i0, rows0, ssem0)
        g_start(i0 + 2, rows0, sem0)
        s_wait(i1, rows1, ssem1)

        @pl.when(g < (NCHUNK - 3) // 2)
        def _():
            g_start(i1 + 2, rows1, sem1)

        return 0

    lax.fori_loop(0, (NCHUNK - 1) // 2, body, 0)
    last = NCHUNK - 1
    g_wait(last, rows0, sem0)
    pltpu.sync_copy(rows0, acc.at[dstbuf.at[last]], add=True)
    plsc.subcore_barrier()
    pltpu.sync_copy(acc.at[pl.ds(s * RPT, RPT)],
                    out_hbm.at[pl.ds(c * ACC_N + s * RPT, RPT)])


def _combine_body(parts_ref, x2_ref, dinv_ref, wt_ref, b_ref, out_ref):
    acc = parts_ref[0] + parts_ref[1] + x2_ref[...]
    pre = acc * dinv_ref[...]
    h = jnp.dot(pre, wt_ref[...], preferred_element_type=jnp.float32)
    o = h + b_ref[...]
    out_ref[...] = jnp.where(o >= 0, o, 0.01 * o) + 1.0


def _combine_call(parts2, x2, dinv, wt, b2):
    BR = 2000
    return pl.pallas_call(
        _combine_body,
        grid=(N // BR,),
        in_specs=[
            pl.BlockSpec((NC, BR, C), lambda i: (0, i, 0)),
            pl.BlockSpec((BR, C), lambda i: (i, 0)),
            pl.BlockSpec((BR, 1), lambda i: (i, 0)),
            pl.BlockSpec((C, C), lambda i: (0, 0)),
            pl.BlockSpec((1, C), lambda i: (0, 0)),
        ],
        out_specs=pl.BlockSpec((BR, C), lambda i: (i, 0)),
        out_shape=jax.ShapeDtypeStruct((N, C), jnp.float32),
    )(parts2, x2, dinv, wt, b2)


def kernel(x, edge_index, W, b):
    src = edge_index[0].astype(jnp.int32)
    dst = edge_index[1].astype(jnp.int32)
    zeros2d = jnp.zeros((RPT, C), jnp.float32)
    zeros1d = jnp.zeros((N,), jnp.float32)

    deg_parts = _deg_kernel(dst, zeros1d).reshape(NW, N)
    x2, dinv = _scale_call(deg_parts.T, x)
    parts = _agg_kernel(src, dst.reshape(NW, NCHUNK, K), x2, zeros2d)
    parts2 = parts.reshape(NC, ACC_N, C)
    out = _combine_call(parts2, x2, dinv, W.T, b.reshape(1, C))
    return out
